# baseline (speedup 1.0000x reference)
import functools
import jax
import jax.numpy as jnp
from jax import lax
from jax.experimental import pallas as pl
from jax.experimental.pallas import tpu as pltpu
from jax.experimental.pallas import tpu_sc as plsc

_B = 16384
_E = 64
_NC = 2
_NS = 16
_NW = _NC * _NS
_BPW = _B // _NW
_PH = 128
_NPH = _BPW // _PH
_TBLK = 4096


def _transpose_pair(xt, n_rows):
    def body(x_ref, y_ref):
        xi = jax.lax.bitcast_convert_type(x_ref[...], jnp.int32)
        t = jnp.transpose(xi)
        y_ref[:, 0:_E] = jax.lax.bitcast_convert_type(t[0:_TBLK // 2, :], jnp.float32)
        y_ref[:, _E:128] = jax.lax.bitcast_convert_type(t[_TBLK // 2:_TBLK, :], jnp.float32)

    nb = pl.cdiv(n_rows, _TBLK)
    return pl.pallas_call(
        body,
        grid=(nb,),
        compiler_params=pltpu.CompilerParams(
            dimension_semantics=("arbitrary",),
        ),
        in_specs=[pl.BlockSpec((_E, _TBLK), lambda b: (0, b))],
        out_specs=pl.BlockSpec((_TBLK // 2, 128), lambda b: (b, 0)),
        out_shape=jax.ShapeDtypeStruct((nb * (_TBLK // 2), 128), jnp.float32),
    )(xt)


def _make_sc_kernel():
    mesh = plsc.VectorSubcoreMesh(core_axis_name="c", subcore_axis_name="s")

    @functools.partial(
        pl.kernel,
        mesh=mesh,
        out_type=jax.ShapeDtypeStruct((_B,), jnp.float32),
        compiler_params=pltpu.CompilerParams(needs_layout_passes=False),
        scratch_types=[
            pltpu.VMEM((_BPW,), jnp.int32),
            pltpu.VMEM((_BPW,), jnp.int32),
            pltpu.VMEM((_BPW,), jnp.int32),
            pltpu.VMEM((_BPW,), jnp.int32),
            pltpu.VMEM((_BPW,), jnp.int32),
            pltpu.VMEM((_BPW,), jnp.int32),
            pltpu.VMEM((_PH, 128), jnp.float32),
            pltpu.VMEM((_PH, 128), jnp.float32),
            pltpu.VMEM((_PH, 128), jnp.float32),
            pltpu.VMEM((_PH, 128), jnp.float32),
            pltpu.VMEM((_PH, 128), jnp.float32),
            pltpu.VMEM((_PH, 128), jnp.float32),
            pltpu.VMEM((_BPW,), jnp.float32),
            pltpu.SemaphoreType.DMA,
            pltpu.SemaphoreType.DMA,
        ],
    )
    def lp_kernel(s_hbm, p_hbm, o_hbm, ent_hbm, rel_hbm, out_hbm,
                  s_idx, p_idx, o_idx, s_pair, p_pair, o_pair,
                  s0, s1, p0, p1, o0, o1, out_v, semA, semB):
        wid = lax.axis_index("s") * _NC + lax.axis_index("c")
        base = wid * _BPW

        ic = [
            pltpu.async_copy(s_hbm.at[pl.ds(base, _BPW)], s_idx, semA),
            pltpu.async_copy(p_hbm.at[pl.ds(base, _BPW)], p_idx, semA),
            pltpu.async_copy(o_hbm.at[pl.ds(base, _BPW)], o_idx, semA),
        ]
        for c in ic:
            c.wait()

        def pair_of(v):
            return lax.bitwise_or(
                lax.shift_left(lax.shift_right_logical(v, 12), 11),
                lax.bitwise_and(v, 2047))

        for k in range(_BPW // 16):
            sl = pl.ds(k * 16, 16)
            s_pair[sl] = pair_of(s_idx[sl])
            p_pair[sl] = pair_of(p_idx[sl])
            o_pair[sl] = pair_of(o_idx[sl])

        sbuf = [s0, s1]
        pbuf = [p0, p1]
        obuf = [o0, o1]
        sems = [semA, semB]
        lane = lax.iota(jnp.int32, 16)

        def fire(ph):
            k = ph % 2
            sl = pl.ds(ph * _PH, _PH)
            sem = sems[k]
            return [
                pltpu.async_copy(ent_hbm.at[s_pair.at[sl]], sbuf[k], sem),
                pltpu.async_copy(rel_hbm.at[p_pair.at[sl]], pbuf[k], sem),
                pltpu.async_copy(ent_hbm.at[o_pair.at[sl]], obuf[k], sem),
            ]

        pend = fire(0)
        for ph in range(_NPH):
            cur = pend
            if ph + 1 < _NPH:
                pend = fire(ph + 1)
            for c in cur:
                c.wait()
            k = ph % 2
            sb, pb, ob = sbuf[k], pbuf[k], obuf[k]

            def chunk_body(ci, carry):
                row_ids = ci * 16 + lane
                gsl = pl.ds(ph * _PH + ci * 16, 16)
                s_par = lax.bitwise_and(lax.shift_right_logical(s_idx[gsl], 11), 1) * _E
                p_par = lax.bitwise_and(lax.shift_right_logical(p_idx[gsl], 11), 1) * _E
                o_par = lax.bitwise_and(lax.shift_right_logical(o_idx[gsl], 11), 1) * _E
                acc = jnp.zeros((16,), jnp.float32)
                for e in range(_E):
                    a = plsc.load_gather(sb, [row_ids, s_par + e])
                    b = plsc.load_gather(pb, [row_ids, p_par + e])
                    c = plsc.load_gather(ob, [row_ids, o_par + e])
                    acc = acc + a * b * c
                out_v[gsl] = acc
                return carry

            lax.fori_loop(0, _PH // 16, chunk_body, 0)

        pltpu.sync_copy(out_v, out_hbm.at[pl.ds(base, _BPW)])

    return lp_kernel


_lp_kernel = None


def kernel(s, p, o, entities, relations):
    global _lp_kernel
    if _lp_kernel is None:
        _lp_kernel = _make_sc_kernel()
    ent2 = _transpose_pair(jnp.swapaxes(entities, 0, 1), entities.shape[0])
    rel2 = _transpose_pair(jnp.swapaxes(relations, 0, 1), relations.shape[0])
    return _lp_kernel(s, p, o, ent2, rel2)


# speedup vs baseline: 1.1186x; 1.1186x over previous
import functools
import jax
import jax.numpy as jnp
from jax import lax
from jax.experimental import pallas as pl
from jax.experimental.pallas import tpu as pltpu
from jax.experimental.pallas import tpu_sc as plsc

_B = 16384
_E = 64
_NC = 2
_NS = 16
_NW = _NC * _NS
_BPW = _B // _NW
_PH = 128
_NPH = _BPW // _PH
_TBLK = 8192


def _transpose_pad(xt, n_rows):
    def body(x_ref, y_ref):
        xi = jax.lax.bitcast_convert_type(x_ref[...], jnp.int32)
        yi = jnp.transpose(xi)
        y_ref[:, 0:_E] = jax.lax.bitcast_convert_type(yi, jnp.float32)

    nb = pl.cdiv(n_rows, _TBLK)
    return pl.pallas_call(
        body,
        grid=(nb,),
        compiler_params=pltpu.CompilerParams(
            dimension_semantics=("arbitrary",),
        ),
        in_specs=[pl.BlockSpec((_E, _TBLK), lambda b: (0, b))],
        out_specs=pl.BlockSpec((_TBLK, 128), lambda b: (b, 0)),
        out_shape=jax.ShapeDtypeStruct((n_rows, 128), jnp.float32),
    )(xt)


def _make_sc_kernel():
    mesh = plsc.VectorSubcoreMesh(core_axis_name="c", subcore_axis_name="s")

    @functools.partial(
        pl.kernel,
        mesh=mesh,
        out_type=jax.ShapeDtypeStruct((_B,), jnp.float32),
        compiler_params=pltpu.CompilerParams(needs_layout_passes=False),
        scratch_types=[
            pltpu.VMEM((_BPW,), jnp.int32),
            pltpu.VMEM((_BPW,), jnp.int32),
            pltpu.VMEM((_BPW,), jnp.int32),
            pltpu.VMEM((_PH, 128), jnp.float32),
            pltpu.VMEM((_PH, 128), jnp.float32),
            pltpu.VMEM((_PH, 128), jnp.float32),
            pltpu.VMEM((_PH, 128), jnp.float32),
            pltpu.VMEM((_PH, 128), jnp.float32),
            pltpu.VMEM((_PH, 128), jnp.float32),
            pltpu.VMEM((_BPW,), jnp.float32),
            pltpu.SemaphoreType.DMA,
            pltpu.SemaphoreType.DMA,
        ],
    )
    def lp_kernel(s_hbm, p_hbm, o_hbm, ent_hbm, rel_hbm, out_hbm,
                  s_idx, p_idx, o_idx, s0, s1, p0, p1, o0, o1,
                  out_v, semA, semB):
        wid = lax.axis_index("s") * _NC + lax.axis_index("c")
        base = wid * _BPW

        ic = [
            pltpu.async_copy(s_hbm.at[pl.ds(base, _BPW)], s_idx, semA),
            pltpu.async_copy(p_hbm.at[pl.ds(base, _BPW)], p_idx, semA),
            pltpu.async_copy(o_hbm.at[pl.ds(base, _BPW)], o_idx, semA),
        ]
        for c in ic:
            c.wait()

        sbuf = [s0, s1]
        pbuf = [p0, p1]
        obuf = [o0, o1]
        sems = [semA, semB]
        lane = lax.iota(jnp.int32, 16)

        def fire(ph):
            k = ph % 2
            sl = pl.ds(ph * _PH, _PH)
            sem = sems[k]
            return [
                pltpu.async_copy(ent_hbm.at[s_idx.at[sl]], sbuf[k], sem),
                pltpu.async_copy(rel_hbm.at[p_idx.at[sl]], pbuf[k], sem),
                pltpu.async_copy(ent_hbm.at[o_idx.at[sl]], obuf[k], sem),
            ]

        pend = fire(0)
        for ph in range(_NPH):
            cur = pend
            if ph + 1 < _NPH:
                pend = fire(ph + 1)
            for c in cur:
                c.wait()
            k = ph % 2
            sb, pb, ob = sbuf[k], pbuf[k], obuf[k]

            def chunk_body(ci, carry):
                row_ids = ci * 16 + lane
                acc = jnp.zeros((16,), jnp.float32)
                for e in range(_E):
                    col = jnp.full((16,), e, dtype=jnp.int32)
                    a = plsc.load_gather(sb, [row_ids, col])
                    b = plsc.load_gather(pb, [row_ids, col])
                    c = plsc.load_gather(ob, [row_ids, col])
                    acc = acc + a * b * c
                out_v[pl.ds(ph * _PH + ci * 16, 16)] = acc
                return carry

            lax.fori_loop(0, _PH // 16, chunk_body, 0)

        pltpu.sync_copy(out_v, out_hbm.at[pl.ds(base, _BPW)])

    return lp_kernel


_lp_kernel = None


def kernel(s, p, o, entities, relations):
    global _lp_kernel
    if _lp_kernel is None:
        _lp_kernel = _make_sc_kernel()
    ent_pad = _transpose_pad(jnp.swapaxes(entities, 0, 1), entities.shape[0])
    rel_pad = _transpose_pad(jnp.swapaxes(relations, 0, 1), relations.shape[0])
    return _lp_kernel(s, p, o, ent_pad, rel_pad)
